# Initial kernel scaffold; baseline (speedup 1.0000x reference)
import jax
import jax.numpy as jnp
from jax.experimental import pallas as pl


def kernel(features, edge_index, W):
    raise NotImplementedError("write your pallas kernel here")


# speedup vs baseline: 7.6284x; 7.6284x over previous
import functools

import jax
import jax.numpy as jnp
from jax import lax
from jax.experimental import pallas as pl
from jax.experimental.pallas import tpu as pltpu
from jax.experimental.pallas import tpu_sc as plsc

N_NODES = 10000
N_EDGES = 160000
D_FEAT = 256
N_CLASSES = 64
K = 10
ALPHA = 0.1

NS = 16
NN = 10240
NPT = NN // NS
CH = 128
EPT = N_EDGES // NS
NCH = (EPT + CH - 1) // CH
EPAD = NCH * CH
DUMMY = N_NODES
D = N_CLASSES
LPR = D // 16
NCC = NPT // CH
OMA = 1.0 - ALPHA


def _matmul_body(x_ref, w_ref, h_ref, ah_ref):
    h = jnp.dot(x_ref[...], w_ref[...], preferred_element_type=jnp.float32)
    h_ref[...] = h
    ah_ref[...] = h * ALPHA


def _project(feat_pad, wt):
    blk = 1024
    return pl.pallas_call(
        _matmul_body,
        grid=(NN // blk,),
        in_specs=[
            pl.BlockSpec((blk, D_FEAT), lambda i: (i, 0)),
            pl.BlockSpec((D_FEAT, D), lambda i: (0, 0)),
        ],
        out_specs=[
            pl.BlockSpec((blk, D), lambda i: (i, 0)),
            pl.BlockSpec((blk, D), lambda i: (i, 0)),
        ],
        out_shape=[
            jax.ShapeDtypeStruct((NN, D), jnp.float32),
            jax.ShapeDtypeStruct((NN, D), jnp.float32),
        ],
    )(feat_pad, wt)


def _norm_body(cs_ref, cd_ref, ns_ref, nd_ref):
    ns_ref[...] = lax.rsqrt(jnp.maximum(cs_ref[...], 1.0))
    nd_ref[...] = OMA * lax.rsqrt(jnp.maximum(cd_ref[...], 1.0))


def _norms(cs, cd):
    return pl.pallas_call(
        _norm_body,
        out_shape=[
            jax.ShapeDtypeStruct(cs.shape, jnp.float32),
            jax.ShapeDtypeStruct(cd.shape, jnp.float32),
        ],
    )(cs, cd)


def _sc_degrees_body(src_hbm, dst_hbm, cs_hbm, cd_hbm,
                     src_v, dst_v, ones16, zbuf16, cs_sh, cd_sh):
    cid = lax.axis_index("c")
    tid = lax.axis_index("s")

    @pl.when(cid == 0)
    def _body():
        row0 = tid * NPT
        zvec = jnp.zeros((16,), jnp.float32)
        ovec = jnp.ones((16,), jnp.float32)

        pltpu.sync_copy(src_hbm.at[tid], src_v)
        pltpu.sync_copy(dst_hbm.at[tid], dst_v)

        def _fill_o(i, _):
            ones16[i, pl.ds(0, 16)] = ovec
            return _
        lax.fori_loop(0, CH, _fill_o, None)

        def _fill_z16(i, _):
            zbuf16[i, pl.ds(0, 16)] = zvec
            return _
        lax.fori_loop(0, NPT, _fill_z16, None)

        pltpu.sync_copy(zbuf16, cs_sh.at[pl.ds(row0, NPT)])
        pltpu.sync_copy(zbuf16, cd_sh.at[pl.ds(row0, NPT)])
        plsc.subcore_barrier()

        def _deg_chunk(j, _):
            pltpu.sync_copy(ones16, cs_sh.at[src_v.at[j]], add=True)
            pltpu.sync_copy(ones16, cd_sh.at[dst_v.at[j]], add=True)
            return _
        lax.fori_loop(0, NCH, _deg_chunk, None)
        plsc.subcore_barrier()

        pltpu.sync_copy(cs_sh.at[pl.ds(row0, NPT)],
                        cs_hbm.at[pl.ds(row0, NPT)])
        pltpu.sync_copy(cd_sh.at[pl.ds(row0, NPT)],
                        cd_hbm.at[pl.ds(row0, NPT)])


def _sc_degrees(srcp, dstp):
    mesh = plsc.VectorSubcoreMesh(core_axis_name="c", subcore_axis_name="s")
    fn = functools.partial(
        pl.kernel,
        mesh=mesh,
        compiler_params=pltpu.CompilerParams(use_tc_tiling_on_sc=False),
        out_type=[
            jax.ShapeDtypeStruct((NN, 16), jnp.float32),
            jax.ShapeDtypeStruct((NN, 16), jnp.float32),
        ],
        scratch_types=[
            pltpu.VMEM((NCH, CH), jnp.int32),
            pltpu.VMEM((NCH, CH), jnp.int32),
            pltpu.VMEM((CH, 16), jnp.float32),
            pltpu.VMEM((NPT, 16), jnp.float32),
            pltpu.VMEM_SHARED((NN, 16), jnp.float32),
            pltpu.VMEM_SHARED((NN, 16), jnp.float32),
        ],
    )(_sc_degrees_body)
    return fn(srcp, dstp)


def _sc_propagate_body(hp_hbm, ah_hbm, src_hbm, dst_hbm, ns_hbm, nd_hbm,
                       out_hbm, xs_hbm,
                       src_v, dst_v, rowbuf, aggc, ahc, outc, zeroc,
                       nsec, ndec, agg_sh, sem):
    cid = lax.axis_index("c")
    tid = lax.axis_index("s")

    @pl.when(cid == 0)
    def _body():
        row0 = tid * NPT
        zvec = jnp.zeros((16,), jnp.float32)

        pltpu.sync_copy(src_hbm.at[tid], src_v)
        pltpu.sync_copy(dst_hbm.at[tid], dst_v)

        def _fill_zc(i, _):
            zeroc[i // LPR, pl.ds((i % LPR) * 16, 16)] = zvec
            return _
        lax.fori_loop(0, CH * LPR, _fill_zc, None)

        def _init_chunk(c, _):
            r0 = row0 + c * CH
            pltpu.sync_copy(zeroc, agg_sh.at[pl.ds(r0, CH)])
            pltpu.sync_copy(hp_hbm.at[pl.ds(r0, CH)], aggc)
            pltpu.sync_copy(ns_hbm.at[pl.ds(r0, CH)], nsec)

            def _rows(r, _):
                ns = nsec[r, pl.ds(0, 16)]
                for v in range(LPR):
                    sl = pl.ds(v * 16, 16)
                    outc[r, sl] = ns * aggc[r, sl]
                return _
            lax.fori_loop(0, CH, _rows, None)
            pltpu.sync_copy(outc, xs_hbm.at[pl.ds(r0, CH)])
            return _
        lax.fori_loop(0, NCC, _init_chunk, None)
        plsc.subcore_barrier()

        def _iter(k, _):
            def _chunk(j, _):
                pltpu.async_copy(xs_hbm.at[src_v.at[j]], rowbuf, sem).wait()
                pltpu.sync_copy(rowbuf, agg_sh.at[dst_v.at[j]], add=True)
                return _
            lax.fori_loop(0, NCH, _chunk, None)
            plsc.subcore_barrier()

            last = k == K - 1

            def _upd_chunk(c, _):
                r0 = row0 + c * CH
                pltpu.sync_copy(agg_sh.at[pl.ds(r0, CH)], aggc)
                pltpu.sync_copy(zeroc, agg_sh.at[pl.ds(r0, CH)])
                pltpu.sync_copy(ah_hbm.at[pl.ds(r0, CH)], ahc)
                pltpu.sync_copy(ns_hbm.at[pl.ds(r0, CH)], nsec)
                pltpu.sync_copy(nd_hbm.at[pl.ds(r0, CH)], ndec)

                def _rows(r, _):
                    ns = nsec[r, pl.ds(0, 16)]
                    nd = ndec[r, pl.ds(0, 16)]
                    for v in range(LPR):
                        sl = pl.ds(v * 16, 16)
                        xv = ahc[r, sl] + nd * aggc[r, sl]
                        outc[r, sl] = jnp.where(last, xv, ns * xv)
                    return _
                lax.fori_loop(0, CH, _rows, None)

                @pl.when(jnp.logical_not(last))
                def _():
                    pltpu.sync_copy(outc, xs_hbm.at[pl.ds(r0, CH)])

                @pl.when(last)
                def _():
                    pltpu.sync_copy(outc, out_hbm.at[pl.ds(r0, CH)])
                return _
            lax.fori_loop(0, NCC, _upd_chunk, None)
            plsc.subcore_barrier()
            return _
        lax.fori_loop(0, K, _iter, None)


def _sc_propagate(hp, ah, srcp, dstp, ns_t, nd_t):
    mesh = plsc.VectorSubcoreMesh(core_axis_name="c", subcore_axis_name="s")
    fn = functools.partial(
        pl.kernel,
        mesh=mesh,
        compiler_params=pltpu.CompilerParams(use_tc_tiling_on_sc=False),
        out_type=[
            jax.ShapeDtypeStruct((NN, D), jnp.float32),
            jax.ShapeDtypeStruct((NN, D), jnp.float32),
        ],
        scratch_types=[
            pltpu.VMEM((NCH, CH), jnp.int32),
            pltpu.VMEM((NCH, CH), jnp.int32),
            pltpu.VMEM((CH, D), jnp.float32),
            pltpu.VMEM((CH, D), jnp.float32),
            pltpu.VMEM((CH, D), jnp.float32),
            pltpu.VMEM((CH, D), jnp.float32),
            pltpu.VMEM((CH, D), jnp.float32),
            pltpu.VMEM((CH, 16), jnp.float32),
            pltpu.VMEM((CH, 16), jnp.float32),
            pltpu.VMEM_SHARED((NN, D), jnp.float32),
            pltpu.SemaphoreType.DMA,
        ],
    )(_sc_propagate_body)
    return fn(hp, ah, srcp, dstp, ns_t, nd_t)


def kernel(features, edge_index, W):
    src = edge_index[0].astype(jnp.int32).reshape(NS, EPT)
    dst = edge_index[1].astype(jnp.int32).reshape(NS, EPT)
    pad = ((0, 0), (0, EPAD - EPT))
    srcp = jnp.pad(src, pad, constant_values=DUMMY).reshape(NS, NCH, CH)
    dstp = jnp.pad(dst, pad, constant_values=DUMMY).reshape(NS, NCH, CH)

    feat_pad = jnp.pad(features, ((0, NN - N_NODES), (0, 0)))
    hp, ah = _project(feat_pad, W.T)

    cs, cd = _sc_degrees(srcp, dstp)
    ns_t, nd_t = _norms(cs, cd)

    out_pad, _ = _sc_propagate(hp, ah, srcp, dstp, ns_t, nd_t)
    return out_pad[:N_NODES]
